# Initial kernel scaffold; baseline (speedup 1.0000x reference)
import functools

import jax
import jax.numpy as jnp
from jax import lax
from jax.experimental import pallas as pl
from jax.experimental.pallas import tpu as pltpu
from jax.experimental.pallas import tpu_sc as plsc

N_USERS = 25000
N_ITEMS = 25000
N = N_USERS + N_ITEMS
D = 64
DH = 32
E = 800000
NLAYERS = 3
NSUB = 16

SLEN = 80
NSTREAM = 5
BLK = SLEN * NSTREAM
EPW = E // NSUB
NB = EPW // BLK
EROWS = E // SLEN

RCH = 125
NCH = N // RCH
NCHS = NCH // NSUB

_mesh = plsc.VectorSubcoreMesh(core_axis_name="c", subcore_axis_name="s")

_f32 = jnp.float32


@functools.partial(
    pl.kernel,
    mesh=_mesh,
    out_type=[
        jax.ShapeDtypeStruct((2 * N, DH), _f32),
        jax.ShapeDtypeStruct((2 * N, DH), _f32),
        jax.ShapeDtypeStruct((2 * N, DH), _f32),
        jax.ShapeDtypeStruct((2 * N, DH), _f32),
    ],
    scratch_types=[
        pltpu.VMEM_SHARED((N, DH), _f32),
        pltpu.VMEM_SHARED((N, 1), _f32),
        pltpu.VMEM((NSTREAM, SLEN), jnp.int32),
        pltpu.VMEM((NSTREAM, SLEN), jnp.int32),
        pltpu.VMEM((NSTREAM, SLEN, DH), _f32),
        pltpu.VMEM((SLEN, 1), _f32),
        pltpu.VMEM((RCH, DH), _f32),
        pltpu.VMEM((RCH, DH), _f32),
        pltpu.VMEM((RCH, 1), _f32),
        pltpu.VMEM((RCH, DH), _f32),
        pltpu.VMEM((RCH, 1), _f32),
    ],
)
def _gcn_sc(xs, src2, dst2, final, t1, t2, accio,
            acc_sh, cnt_sh, sidx, didx, rows, ones, xbuf, abuf, cbuf,
            zbuf, zcbuf):
    cidx = lax.axis_index("c")
    sid = lax.axis_index("s")
    cbase = cidx * N

    zvec = jnp.zeros((16,), _f32)

    @pl.loop(0, SLEN)
    def _(i):
        ones[i, 0] = 1.0

    @pl.loop(0, RCH)
    def _(r):
        for v in range(DH // 16):
            zbuf[r, pl.ds(v * 16, 16)] = zvec
        zcbuf[r, 0] = 0.0

    def zero_phase(first):
        @pl.loop(0, NCHS)
        def _(j):
            r0 = (j * NSUB + sid) * RCH
            pltpu.sync_copy(zbuf, acc_sh.at[pl.ds(r0, RCH)])
            if first:
                pltpu.sync_copy(zcbuf, cnt_sh.at[pl.ds(r0, RCH)])

    def scatter_phase(tbl, first):
        @pl.loop(0, NB)
        def _(b):
            r0 = sid * (EPW // SLEN) + b * NSTREAM
            pltpu.sync_copy(src2.at[pl.ds(r0, NSTREAM)], sidx)
            pltpu.sync_copy(dst2.at[pl.ds(r0, NSTREAM)], didx)
            for k in range(NSTREAM):
                for v in range(SLEN // 16):
                    sl = pl.ds(v * 16, 16)
                    sidx[k, sl] = sidx[k, sl] + cbase
            for k in range(NSTREAM):
                pltpu.sync_copy(tbl.at[sidx.at[k]], rows.at[k])
                pltpu.sync_copy(rows.at[k], acc_sh.at[didx.at[k]], add=True)
                if first:
                    pltpu.sync_copy(ones, cnt_sh.at[didx.at[k]], add=True)

    def norm_phase(prev, nxt, last):
        @pl.loop(0, NCHS)
        def _(j):
            r0 = (j * NSUB + sid) * RCH
            pltpu.sync_copy(acc_sh.at[pl.ds(r0, RCH)], xbuf)
            pltpu.sync_copy(cnt_sh.at[pl.ds(r0, RCH)], cbuf)
            pltpu.sync_copy(prev.at[pl.ds(cbase + r0, RCH)], abuf)

            @pl.loop(0, RCH)
            def _(r):
                rec = 1.0 / jnp.maximum(cbuf[r, 0], 1.0)
                for v in range(DH // 16):
                    sl = pl.ds(v * 16, 16)
                    xv = xbuf[r, sl] * rec
                    av = abuf[r, sl] + xv
                    if last:
                        abuf[r, sl] = av * 0.25
                    else:
                        xbuf[r, sl] = xv
                        abuf[r, sl] = av

            if last:
                pltpu.sync_copy(abuf, final.at[pl.ds(cbase + r0, RCH)])
            else:
                pltpu.sync_copy(xbuf, nxt.at[pl.ds(cbase + r0, RCH)])
                pltpu.sync_copy(abuf, accio.at[pl.ds(cbase + r0, RCH)])

    tbls = [xs, t1, t2]
    prevs = [xs, accio, accio]
    for l in range(NLAYERS):
        zero_phase(first=(l == 0))
        plsc.subcore_barrier()
        scatter_phase(tbls[l], first=(l == 0))
        plsc.subcore_barrier()
        norm_phase(prevs[l], tbls[l + 1] if l < NLAYERS - 1 else None,
                   last=(l == NLAYERS - 1))
        plsc.subcore_barrier()


def kernel(user_weight, item_weight, edge_index):
    x = jnp.concatenate([user_weight, item_weight], axis=0)
    xs = jnp.concatenate([x[:, :DH], x[:, DH:]], axis=0)
    src2 = edge_index[0].astype(jnp.int32).reshape(EROWS, SLEN)
    dst2 = edge_index[1].astype(jnp.int32).reshape(EROWS, SLEN)
    final, _t1, _t2, _acc = _gcn_sc(xs, src2, dst2)
    fe = jnp.concatenate([final[:N], final[N:]], axis=1)
    return fe[:N_USERS], fe[N_USERS:]


# speedup vs baseline: 4.2300x; 4.2300x over previous
import functools

import jax
import jax.numpy as jnp
from jax import lax
from jax.experimental import pallas as pl
from jax.experimental.pallas import tpu as pltpu
from jax.experimental.pallas import tpu_sc as plsc

N_USERS = 25000
N_ITEMS = 25000
N = N_USERS + N_ITEMS
D = 64
DH = 32
E = 800000
NLAYERS = 3
NSUB = 16

SLEN = 128
NSTREAM = 8
BLK = SLEN * NSTREAM
EPAD = 819200
EROWS = EPAD // SLEN
NBPW = EROWS // NSTREAM // NSUB

RCH = 200
NCH = N // RCH
NCHS = -(-NCH // NSUB)

_mesh = plsc.VectorSubcoreMesh(core_axis_name="c", subcore_axis_name="s")

_f32 = jnp.float32


@functools.partial(
    pl.kernel,
    mesh=_mesh,
    compiler_params=pltpu.CompilerParams(use_tc_tiling_on_sc=False),
    out_type=[
        jax.ShapeDtypeStruct((2 * N, DH), _f32),
        jax.ShapeDtypeStruct((2 * N, DH), _f32),
        jax.ShapeDtypeStruct((2 * N, DH), _f32),
        jax.ShapeDtypeStruct((2 * N, DH), _f32),
        jax.ShapeDtypeStruct((2 * N, 16), _f32),
    ],
    scratch_types=[
        pltpu.VMEM_SHARED((N + 8, DH), _f32),
        pltpu.VMEM((NSTREAM, SLEN), jnp.int32),
        pltpu.VMEM((NSTREAM, SLEN), jnp.int32),
        pltpu.VMEM((SLEN, DH), _f32),
        pltpu.VMEM((RCH, DH), _f32),
        pltpu.VMEM((RCH, DH), _f32),
        pltpu.VMEM((RCH, 16), _f32),
    ],
)
def _gcn_sc(xs, src2, dst2, final, t1, t2, accio, rec,
            acc_sh, sidx, didx, rows, xbuf, abuf, rbuf):
    cidx = lax.axis_index("c")
    sid = lax.axis_index("s")
    cbase = cidx * N

    zvec = jnp.zeros((16,), _f32)
    onev = jnp.ones((16,), _f32)

    def fill(ref, n, vec):
        @pl.loop(0, n)
        def _(r):
            for v in range(DH // 16):
                ref[r, pl.ds(v * 16, 16)] = vec

    def chunk_loop(body):
        @pl.loop(0, NCHS)
        def _(j):
            g = j * NSUB + sid

            @pl.when(g < NCH)
            def _():
                body(j, g * RCH)

    def zero_phase():
        fill(abuf, RCH, zvec)
        chunk_loop(lambda j, r0:
                   pltpu.sync_copy(abuf, acc_sh.at[pl.ds(r0, RCH)]))

    def count_phase():
        fill(rows, SLEN, onev)

        @pl.loop(0, NBPW)
        def _(b):
            r0 = (sid * NBPW + b) * NSTREAM
            pltpu.sync_copy(dst2.at[pl.ds(r0, NSTREAM)], didx)
            for k in range(NSTREAM):
                pltpu.sync_copy(rows, acc_sh.at[didx.at[k]], add=True)

    def extract_phase():
        def body(j, r0):
            pltpu.sync_copy(acc_sh.at[pl.ds(r0, RCH)], xbuf)

            @pl.loop(0, RCH)
            def _(r):
                cv = xbuf[r, pl.ds(0, 16)]
                rbuf[r, pl.ds(0, 16)] = 1.0 / jnp.maximum(cv, 1.0)

            pltpu.sync_copy(rbuf, rec.at[pl.ds(cbase + r0, RCH)])

        chunk_loop(body)

    def scatter_phase(tbl):
        @pl.loop(0, NBPW)
        def _(b):
            r0 = (sid * NBPW + b) * NSTREAM
            pltpu.sync_copy(src2.at[pl.ds(r0, NSTREAM)], sidx)
            pltpu.sync_copy(dst2.at[pl.ds(r0, NSTREAM)], didx)
            for k in range(NSTREAM):
                for v in range(SLEN // 16):
                    sl = pl.ds(v * 16, 16)
                    sidx[k, sl] = sidx[k, sl] + cbase
            for k in range(NSTREAM):
                pltpu.sync_copy(tbl.at[sidx.at[k]], rows)
                pltpu.sync_copy(rows, acc_sh.at[didx.at[k]], add=True)

    def norm_phase(prev, nxt, last):
        def body(j, r0):
            pltpu.sync_copy(acc_sh.at[pl.ds(r0, RCH)], xbuf)
            pltpu.sync_copy(prev.at[pl.ds(cbase + r0, RCH)], abuf)
            pltpu.sync_copy(rec.at[pl.ds(cbase + r0, RCH)], rbuf)

            @pl.loop(0, RCH)
            def _(r):
                recv = rbuf[r, pl.ds(0, 16)]
                for v in range(DH // 16):
                    sl = pl.ds(v * 16, 16)
                    xv = xbuf[r, sl] * recv
                    av = abuf[r, sl] + xv
                    if last:
                        abuf[r, sl] = av * 0.25
                    else:
                        xbuf[r, sl] = xv
                        abuf[r, sl] = av

            if last:
                pltpu.sync_copy(abuf, final.at[pl.ds(cbase + r0, RCH)])
            else:
                pltpu.sync_copy(xbuf, nxt.at[pl.ds(cbase + r0, RCH)])
                pltpu.sync_copy(abuf, accio.at[pl.ds(cbase + r0, RCH)])

        chunk_loop(body)

    zero_phase()
    plsc.subcore_barrier()
    count_phase()
    plsc.subcore_barrier()
    extract_phase()
    zero_phase()
    plsc.subcore_barrier()

    tbls = [xs, t1, t2]
    prevs = [xs, accio, accio]
    for l in range(NLAYERS):
        scatter_phase(tbls[l])
        plsc.subcore_barrier()
        norm_phase(prevs[l], tbls[l + 1] if l < NLAYERS - 1 else None,
                   last=(l == NLAYERS - 1))
        if l < NLAYERS - 1:
            zero_phase()
            plsc.subcore_barrier()


def kernel(user_weight, item_weight, edge_index):
    x = jnp.concatenate([user_weight, item_weight], axis=0)
    xs = jnp.concatenate([x[:, :DH], x[:, DH:]], axis=0)
    src = edge_index[0].astype(jnp.int32)
    dst = edge_index[1].astype(jnp.int32)
    pad = EPAD - E
    src2 = jnp.concatenate([src, jnp.zeros((pad,), jnp.int32)])
    dst2 = jnp.concatenate([dst, jnp.full((pad,), N, jnp.int32)])
    final, _t1, _t2, _acc, _rec = _gcn_sc(
        xs, src2.reshape(EROWS, SLEN), dst2.reshape(EROWS, SLEN))
    fe = jnp.concatenate([final[:N], final[N:]], axis=1)
    return fe[:N_USERS], fe[N_USERS:]


# speedup vs baseline: 5.1629x; 1.2206x over previous
import functools

import jax
import jax.numpy as jnp
from jax import lax
from jax.experimental import pallas as pl
from jax.experimental.pallas import tpu as pltpu
from jax.experimental.pallas import tpu_sc as plsc

N_USERS = 25000
N_ITEMS = 25000
N = N_USERS + N_ITEMS
D = 64
DH = 32
E = 800000
NLAYERS = 3
NSUB = 16

SLEN = 128
NSTREAM = 8
BLK = SLEN * NSTREAM
EPAD = 819200
EROWS = EPAD // SLEN
NBPW = EROWS // NSTREAM // NSUB

RCH = 200
NCH = N // RCH
NCHS = -(-NCH // NSUB)

_mesh = plsc.VectorSubcoreMesh(core_axis_name="c", subcore_axis_name="s")

_f32 = jnp.float32


@functools.partial(
    pl.kernel,
    mesh=_mesh,
    compiler_params=pltpu.CompilerParams(use_tc_tiling_on_sc=False),
    out_type=[
        jax.ShapeDtypeStruct((2 * N, DH), _f32),
        jax.ShapeDtypeStruct((2 * N, DH), _f32),
        jax.ShapeDtypeStruct((2 * N, DH), _f32),
        jax.ShapeDtypeStruct((2 * N, DH), _f32),
        jax.ShapeDtypeStruct((2 * N, 16), _f32),
    ],
    scratch_types=[
        pltpu.VMEM_SHARED((N + 8, DH), _f32),
        pltpu.VMEM((NSTREAM, SLEN), jnp.int32),
        pltpu.VMEM((NSTREAM, SLEN), jnp.int32),
        pltpu.VMEM((SLEN, DH), _f32),
        pltpu.VMEM((SLEN, DH), _f32),
        pltpu.VMEM((RCH, DH), _f32),
        pltpu.VMEM((RCH, DH), _f32),
        pltpu.VMEM((RCH, 16), _f32),
        pltpu.SemaphoreType.DMA,
        pltpu.SemaphoreType.DMA,
    ],
)
def _gcn_sc(xs, src2, dst2, final, t1, t2, accio, rec,
            acc_sh, sidx, didx, rows, rows_b, xbuf, abuf, rbuf,
            sem_a, sem_b):
    cidx = lax.axis_index("c")
    sid = lax.axis_index("s")
    cbase = cidx * N

    zvec = jnp.zeros((16,), _f32)
    onev = jnp.ones((16,), _f32)

    def fill(ref, n, vec):
        @pl.loop(0, n)
        def _(r):
            for v in range(DH // 16):
                ref[r, pl.ds(v * 16, 16)] = vec

    def chunk_loop(body):
        @pl.loop(0, NCHS)
        def _(j):
            g = j * NSUB + sid

            @pl.when(g < NCH)
            def _():
                body(j, g * RCH)

    def zero_phase():
        fill(abuf, RCH, zvec)
        chunk_loop(lambda j, r0:
                   pltpu.sync_copy(abuf, acc_sh.at[pl.ds(r0, RCH)]))

    def count_phase():
        fill(rows, SLEN, onev)

        @pl.loop(0, NBPW)
        def _(b):
            r0 = (sid * NBPW + b) * NSTREAM
            pltpu.sync_copy(dst2.at[pl.ds(r0, NSTREAM)], didx)
            for k in range(NSTREAM):
                pltpu.sync_copy(rows, acc_sh.at[didx.at[k]], add=True)

    def extract_phase():
        def body(j, r0):
            pltpu.sync_copy(acc_sh.at[pl.ds(r0, RCH)], xbuf)

            @pl.loop(0, RCH)
            def _(r):
                cv = xbuf[r, pl.ds(0, 16)]
                rbuf[r, pl.ds(0, 16)] = 1.0 / jnp.maximum(cv, 1.0)

            pltpu.sync_copy(rbuf, rec.at[pl.ds(cbase + r0, RCH)])

        chunk_loop(body)

    def scatter_phase(tbl):
        @pl.loop(0, NBPW)
        def _(b):
            r0 = (sid * NBPW + b) * NSTREAM
            pltpu.sync_copy(src2.at[pl.ds(cidx * EROWS + r0, NSTREAM)], sidx)
            pltpu.sync_copy(dst2.at[pl.ds(r0, NSTREAM)], didx)
            bufs = (rows, rows_b)
            sems = (sem_a, sem_b)
            cp = pltpu.async_copy(tbl.at[sidx.at[0]], bufs[0], sems[0])
            for k in range(NSTREAM):
                nk = (k + 1) % 2
                if k + 1 < NSTREAM:
                    cpn = pltpu.async_copy(tbl.at[sidx.at[k + 1]],
                                           bufs[nk], sems[nk])
                cp.wait()
                pltpu.sync_copy(bufs[k % 2], acc_sh.at[didx.at[k]], add=True)
                if k + 1 < NSTREAM:
                    cp = cpn

    def norm_phase(prev, nxt, last):
        def body(j, r0):
            cp1 = pltpu.async_copy(prev.at[pl.ds(cbase + r0, RCH)], abuf,
                                   sem_a)
            cp2 = pltpu.async_copy(rec.at[pl.ds(cbase + r0, RCH)], rbuf,
                                   sem_b)
            pltpu.sync_copy(acc_sh.at[pl.ds(r0, RCH)], xbuf)
            cp1.wait()
            cp2.wait()

            @pl.loop(0, RCH)
            def _(r):
                recv = rbuf[r, pl.ds(0, 16)]
                for v in range(DH // 16):
                    sl = pl.ds(v * 16, 16)
                    xv = xbuf[r, sl] * recv
                    av = abuf[r, sl] + xv
                    if last:
                        abuf[r, sl] = av * 0.25
                    else:
                        xbuf[r, sl] = xv
                        abuf[r, sl] = av

            if last:
                pltpu.sync_copy(abuf, final.at[pl.ds(cbase + r0, RCH)])
            else:
                pltpu.sync_copy(xbuf, nxt.at[pl.ds(cbase + r0, RCH)])
                pltpu.sync_copy(abuf, accio.at[pl.ds(cbase + r0, RCH)])

        chunk_loop(body)

    zero_phase()
    plsc.subcore_barrier()
    count_phase()
    plsc.subcore_barrier()
    extract_phase()
    zero_phase()
    plsc.subcore_barrier()

    tbls = [xs, t1, t2]
    prevs = [xs, accio, accio]
    for l in range(NLAYERS):
        scatter_phase(tbls[l])
        plsc.subcore_barrier()
        norm_phase(prevs[l], tbls[l + 1] if l < NLAYERS - 1 else None,
                   last=(l == NLAYERS - 1))
        if l < NLAYERS - 1:
            zero_phase()
            plsc.subcore_barrier()


def kernel(user_weight, item_weight, edge_index):
    x = jnp.concatenate([user_weight, item_weight], axis=0)
    xs = jnp.concatenate([x[:, :DH], x[:, DH:]], axis=0)
    src = edge_index[0].astype(jnp.int32)
    dst = edge_index[1].astype(jnp.int32)
    pad = EPAD - E
    src2 = jnp.concatenate([src, jnp.zeros((pad,), jnp.int32)])
    dst2 = jnp.concatenate([dst, jnp.full((pad,), N, jnp.int32)])
    src2 = jnp.concatenate([src2.reshape(EROWS, SLEN),
                            src2.reshape(EROWS, SLEN) + N])
    final, _t1, _t2, _acc, _rec = _gcn_sc(
        xs, src2, dst2.reshape(EROWS, SLEN))
    fe = jnp.concatenate([final[:N], final[N:]], axis=1)
    return fe[:N_USERS], fe[N_USERS:]


# speedup vs baseline: 5.4780x; 1.0610x over previous
import functools

import jax
import jax.numpy as jnp
from jax import lax
from jax.experimental import pallas as pl
from jax.experimental.pallas import tpu as pltpu
from jax.experimental.pallas import tpu_sc as plsc

N_USERS = 25000
N_ITEMS = 25000
N = N_USERS + N_ITEMS
D = 64
DH = 32
E = 800000
NLAYERS = 3
NSUB = 16

SLEN = 128
NSTREAM = 8
BLK = SLEN * NSTREAM
EPAD = 819200
EROWS = EPAD // SLEN
NBPW = EROWS // NSTREAM // NSUB

RCH = 125
NCH = N // RCH
NCHS = NCH // NSUB

_mesh = plsc.VectorSubcoreMesh(core_axis_name="c", subcore_axis_name="s")

_f32 = jnp.float32


@functools.partial(
    pl.kernel,
    mesh=_mesh,
    compiler_params=pltpu.CompilerParams(use_tc_tiling_on_sc=False),
    out_type=[
        jax.ShapeDtypeStruct((2 * N, DH), _f32),
        jax.ShapeDtypeStruct((2 * N, DH), _f32),
        jax.ShapeDtypeStruct((2 * N, DH), _f32),
        jax.ShapeDtypeStruct((2 * N, DH), _f32),
        jax.ShapeDtypeStruct((2 * N, 16), _f32),
    ],
    scratch_types=[
        pltpu.VMEM_SHARED((N + 8, DH), _f32),
        pltpu.VMEM((NSTREAM, SLEN), jnp.int32),
        pltpu.VMEM((NSTREAM, SLEN), jnp.int32),
        pltpu.VMEM((SLEN, DH), _f32),
        pltpu.VMEM((SLEN, DH), _f32),
        pltpu.VMEM((SLEN, DH), _f32),
        pltpu.VMEM((RCH, DH), _f32),
        pltpu.VMEM((RCH, DH), _f32),
        pltpu.VMEM((RCH, 16), _f32),
        pltpu.SemaphoreType.DMA,
        pltpu.SemaphoreType.DMA,
        pltpu.SemaphoreType.DMA,
        pltpu.SemaphoreType.DMA,
        pltpu.SemaphoreType.DMA,
        pltpu.SemaphoreType.DMA,
        pltpu.SemaphoreType.DMA,
        pltpu.SemaphoreType.DMA,
    ],
)
def _gcn_sc(xs, src2, dst2, final, t1, t2, accio, rec,
            acc_sh, sidx, didx, rows, rows_1, rows_2, xbuf, abuf, rbuf,
            sem_a, sem_b, gs0, gs1, gs2, ss0, ss1, ss2):
    cidx = lax.axis_index("c")
    sid = lax.axis_index("s")
    cbase = cidx * N

    zvec = jnp.zeros((16,), _f32)
    onev = jnp.ones((16,), _f32)

    def fill(ref, n, vec):
        @pl.loop(0, n)
        def _(r):
            for v in range(DH // 16):
                ref[r, pl.ds(v * 16, 16)] = vec

    def chunk_loop(body):
        @pl.loop(0, NCHS)
        def _(j):
            g = j * NSUB + sid

            @pl.when(g < NCH)
            def _():
                body(j, g * RCH)

    def zero_phase():
        fill(abuf, RCH, zvec)
        chunk_loop(lambda j, r0:
                   pltpu.sync_copy(abuf, acc_sh.at[pl.ds(r0, RCH)]))

    ssems = (ss0, ss1, ss2)
    gsems = (gs0, gs1, gs2)

    def count_phase():
        fill(rows, SLEN, onev)

        @pl.loop(0, NBPW)
        def _(b):
            r0 = (sid * NBPW + b) * NSTREAM
            pltpu.sync_copy(dst2.at[pl.ds(r0, NSTREAM)], didx)
            ss = [pltpu.async_copy(rows, acc_sh.at[didx.at[k]],
                                   ssems[k % 3], add=True)
                  for k in range(NSTREAM)]
            for s in ss:
                s.wait()

    def extract_phase():
        def body(j, r0):
            pltpu.sync_copy(acc_sh.at[pl.ds(r0, RCH)], xbuf)

            @pl.loop(0, RCH)
            def _(r):
                cv = xbuf[r, pl.ds(0, 16)]
                rbuf[r, pl.ds(0, 16)] = 1.0 / jnp.maximum(cv, 1.0)

            pltpu.sync_copy(rbuf, rec.at[pl.ds(cbase + r0, RCH)])

        chunk_loop(body)

    def scatter_phase(tbl):
        bufs = (rows, rows_1, rows_2)

        @pl.loop(0, NBPW)
        def _(b):
            r0 = (sid * NBPW + b) * NSTREAM
            ci = pltpu.async_copy(src2.at[pl.ds(cidx * EROWS + r0, NSTREAM)],
                                  sidx, sem_a)
            cd = pltpu.async_copy(dst2.at[pl.ds(r0, NSTREAM)], didx, sem_b)
            ci.wait()
            cd.wait()
            g = [None] * NSTREAM
            s = [None] * NSTREAM
            for j in range(2):
                g[j] = pltpu.async_copy(tbl.at[sidx.at[j]], bufs[j], gsems[j])
            for k in range(NSTREAM):
                g[k].wait()
                s[k] = pltpu.async_copy(bufs[k % 3], acc_sh.at[didx.at[k]],
                                        ssems[k % 3], add=True)
                j = k + 2
                if j < NSTREAM:
                    if j >= 3:
                        s[j - 3].wait()
                    g[j] = pltpu.async_copy(tbl.at[sidx.at[j]], bufs[j % 3],
                                            gsems[j % 3])
            for k in range(NSTREAM - 3, NSTREAM):
                s[k].wait()

    def norm_phase(prev, nxt, last):
        def body(j, r0):
            cp1 = pltpu.async_copy(prev.at[pl.ds(cbase + r0, RCH)], abuf,
                                   sem_a)
            cp2 = pltpu.async_copy(rec.at[pl.ds(cbase + r0, RCH)], rbuf,
                                   sem_b)
            pltpu.sync_copy(acc_sh.at[pl.ds(r0, RCH)], xbuf)
            cp1.wait()
            cp2.wait()

            @pl.loop(0, RCH)
            def _(r):
                recv = rbuf[r, pl.ds(0, 16)]
                for v in range(DH // 16):
                    sl = pl.ds(v * 16, 16)
                    xv = xbuf[r, sl] * recv
                    av = abuf[r, sl] + xv
                    if last:
                        abuf[r, sl] = av * 0.25
                    else:
                        xbuf[r, sl] = xv
                        abuf[r, sl] = av

            if last:
                pltpu.sync_copy(abuf, final.at[pl.ds(cbase + r0, RCH)])
            else:
                pltpu.sync_copy(xbuf, nxt.at[pl.ds(cbase + r0, RCH)])
                pltpu.sync_copy(abuf, accio.at[pl.ds(cbase + r0, RCH)])

        chunk_loop(body)

    zero_phase()
    plsc.subcore_barrier()
    count_phase()
    plsc.subcore_barrier()
    extract_phase()
    zero_phase()
    plsc.subcore_barrier()

    tbls = [xs, t1, t2]
    prevs = [xs, accio, accio]
    for l in range(NLAYERS):
        scatter_phase(tbls[l])
        plsc.subcore_barrier()
        norm_phase(prevs[l], tbls[l + 1] if l < NLAYERS - 1 else None,
                   last=(l == NLAYERS - 1))
        if l < NLAYERS - 1:
            zero_phase()
            plsc.subcore_barrier()


def kernel(user_weight, item_weight, edge_index):
    x = jnp.concatenate([user_weight, item_weight], axis=0)
    xs = jnp.concatenate([x[:, :DH], x[:, DH:]], axis=0)
    src = edge_index[0].astype(jnp.int32)
    dst = edge_index[1].astype(jnp.int32)
    pad = EPAD - E
    src2 = jnp.concatenate([src, jnp.zeros((pad,), jnp.int32)])
    dst2 = jnp.concatenate([dst, jnp.full((pad,), N, jnp.int32)])
    src2 = jnp.concatenate([src2.reshape(EROWS, SLEN),
                            src2.reshape(EROWS, SLEN) + N])
    final, _t1, _t2, _acc, _rec = _gcn_sc(
        xs, src2, dst2.reshape(EROWS, SLEN))
    fe = jnp.concatenate([final[:N], final[N:]], axis=1)
    return fe[:N_USERS], fe[N_USERS:]


# speedup vs baseline: 7.4364x; 1.3575x over previous
import functools

import jax
import jax.numpy as jnp
from jax import lax
from jax.experimental import pallas as pl
from jax.experimental.pallas import tpu as pltpu
from jax.experimental.pallas import tpu_sc as plsc

N_USERS = 25000
N_ITEMS = 25000
N = N_USERS + N_ITEMS
D = 64
DQ = 16
E = 800000
NLAYERS = 3
NSUB = 16

SLEN = 128
NSTREAM = 8
EPAD = 819200
EROWS = EPAD // SLEN
NBPW = EROWS // NSTREAM // NSUB

RCH = 125
NCH = N // RCH
NCHS = NCH // NSUB
LDW = N // NSUB

_mesh = plsc.VectorSubcoreMesh(core_axis_name="c", subcore_axis_name="s")

_f32 = jnp.float32


@functools.partial(
    pl.kernel,
    mesh=_mesh,
    compiler_params=pltpu.CompilerParams(use_tc_tiling_on_sc=False),
    out_type=[
        jax.ShapeDtypeStruct((4 * N, DQ), _f32),
        jax.ShapeDtypeStruct((4 * N, DQ), _f32),
        jax.ShapeDtypeStruct((4 * N, DQ), _f32),
        jax.ShapeDtypeStruct((4 * N, DQ), _f32),
        jax.ShapeDtypeStruct((2 * N, DQ), _f32),
    ],
    scratch_types=[
        pltpu.VMEM_SHARED((N, DQ), _f32),
        pltpu.VMEM_SHARED((N + 8, DQ), _f32),
        pltpu.VMEM((NSTREAM, SLEN), jnp.int32),
        pltpu.VMEM((NSTREAM, SLEN), jnp.int32),
        pltpu.VMEM((SLEN, DQ), _f32),
        pltpu.VMEM((SLEN, DQ), _f32),
        pltpu.VMEM((SLEN, DQ), _f32),
        pltpu.VMEM((RCH, DQ), _f32),
        pltpu.VMEM((RCH, DQ), _f32),
        pltpu.VMEM((RCH, DQ), _f32),
        pltpu.SemaphoreType.DMA,
        pltpu.SemaphoreType.DMA,
        pltpu.SemaphoreType.DMA,
        pltpu.SemaphoreType.DMA,
        pltpu.SemaphoreType.DMA,
        pltpu.SemaphoreType.DMA,
        pltpu.SemaphoreType.DMA,
        pltpu.SemaphoreType.DMA,
    ],
)
def _gcn_sc(src2, dst2, x0q, final, t1, t2, accio, rec,
            tbl_sh, acc_sh, sidx, didx, rows, rows_1, rows_2,
            xbuf, abuf, rbuf, sem_a, sem_b, gs0, gs1, gs2, ss0, ss1, ss2):
    cidx = lax.axis_index("c")
    sid = lax.axis_index("s")
    cbase = cidx * N
    qbase0 = 2 * cidx * N

    zvec = jnp.zeros((16,), _f32)
    onev = jnp.ones((16,), _f32)
    gsems = (gs0, gs1, gs2)
    ssems = (ss0, ss1, ss2)
    bufs = (rows, rows_1, rows_2)

    def fill(ref, n, vec):
        @pl.loop(0, n)
        def _(r):
            ref[r, pl.ds(0, 16)] = vec

    def chunk_loop(body):
        @pl.loop(0, NCHS)
        def _(j):
            body(j, (j * NSUB + sid) * RCH)

    def zero_phase():
        fill(abuf, RCH, zvec)
        chunk_loop(lambda j, r0:
                   pltpu.sync_copy(abuf, acc_sh.at[pl.ds(r0, RCH)]))

    def load_phase(src, qb):
        r0 = sid * LDW
        pltpu.sync_copy(src.at[pl.ds(qb + r0, LDW)],
                        tbl_sh.at[pl.ds(r0, LDW)])

    def count_phase():
        fill(rows, SLEN, onev)

        @pl.loop(0, NBPW)
        def _(b):
            r0 = (sid * NBPW + b) * NSTREAM
            pltpu.sync_copy(dst2.at[pl.ds(r0, NSTREAM)], didx)
            ss = [pltpu.async_copy(rows, acc_sh.at[didx.at[k]],
                                   ssems[k % 3], add=True)
                  for k in range(NSTREAM)]
            for s in ss:
                s.wait()

    def extract_phase():
        def body(j, r0):
            pltpu.sync_copy(acc_sh.at[pl.ds(r0, RCH)], xbuf)

            @pl.loop(0, RCH)
            def _(r):
                cv = xbuf[r, pl.ds(0, 16)]
                rbuf[r, pl.ds(0, 16)] = 1.0 / jnp.maximum(cv, 1.0)

            pltpu.sync_copy(rbuf, rec.at[pl.ds(cbase + r0, RCH)])

        chunk_loop(body)

    def stream_phase():
        @pl.loop(0, NBPW)
        def _(b):
            r0 = (sid * NBPW + b) * NSTREAM
            ci = pltpu.async_copy(src2.at[pl.ds(r0, NSTREAM)], sidx, sem_a)
            cd = pltpu.async_copy(dst2.at[pl.ds(r0, NSTREAM)], didx, sem_b)
            ci.wait()
            cd.wait()
            g = [None] * NSTREAM
            s = [None] * NSTREAM
            for j in range(2):
                g[j] = pltpu.async_copy(tbl_sh.at[sidx.at[j]], bufs[j],
                                        gsems[j])
            for k in range(NSTREAM):
                g[k].wait()
                s[k] = pltpu.async_copy(bufs[k % 3], acc_sh.at[didx.at[k]],
                                        ssems[k % 3], add=True)
                j = k + 2
                if j < NSTREAM:
                    if j >= 3:
                        s[j - 3].wait()
                    g[j] = pltpu.async_copy(tbl_sh.at[sidx.at[j]],
                                            bufs[j % 3], gsems[j % 3])
            for k in range(NSTREAM - 3, NSTREAM):
                s[k].wait()

    def norm_phase(prev, nxt, qb, last):
        def body(j, r0):
            cp1 = pltpu.async_copy(prev.at[pl.ds(qb + r0, RCH)], abuf, sem_a)
            cp2 = pltpu.async_copy(rec.at[pl.ds(cbase + r0, RCH)], rbuf,
                                   sem_b)
            pltpu.sync_copy(acc_sh.at[pl.ds(r0, RCH)], xbuf)
            cp1.wait()
            cp2.wait()

            @pl.loop(0, RCH)
            def _(r):
                sl = pl.ds(0, 16)
                xv = xbuf[r, sl] * rbuf[r, sl]
                av = abuf[r, sl] + xv
                if last:
                    abuf[r, sl] = av * 0.25
                else:
                    xbuf[r, sl] = xv
                    abuf[r, sl] = av

            if last:
                pltpu.sync_copy(abuf, final.at[pl.ds(qb + r0, RCH)])
            else:
                pltpu.sync_copy(xbuf, nxt.at[pl.ds(qb + r0, RCH)])
                pltpu.sync_copy(abuf, accio.at[pl.ds(qb + r0, RCH)])

        chunk_loop(body)

    zero_phase()
    plsc.subcore_barrier()
    count_phase()
    plsc.subcore_barrier()
    extract_phase()

    tbls = [x0q, t1, t2]
    prevs = [x0q, accio, accio]
    for l in range(NLAYERS):
        for q in range(2):
            qb = qbase0 + q * N
            load_phase(tbls[l], qb)
            zero_phase()
            plsc.subcore_barrier()
            stream_phase()
            plsc.subcore_barrier()
            norm_phase(prevs[l], tbls[l + 1] if l < NLAYERS - 1 else None,
                       qb, last=(l == NLAYERS - 1))


def kernel(user_weight, item_weight, edge_index):
    x = jnp.concatenate([user_weight, item_weight], axis=0)
    x0q = jnp.concatenate([x[:, 0:16], x[:, 16:32], x[:, 32:48], x[:, 48:64]],
                          axis=0)
    src = edge_index[0].astype(jnp.int32)
    dst = edge_index[1].astype(jnp.int32)
    pad = EPAD - E
    src2 = jnp.concatenate([src, jnp.zeros((pad,), jnp.int32)])
    dst2 = jnp.concatenate([dst, jnp.full((pad,), N, jnp.int32)])
    final, _t1, _t2, _acc, _rec = _gcn_sc(
        src2.reshape(EROWS, SLEN), dst2.reshape(EROWS, SLEN), x0q)
    fe = jnp.concatenate([final[0:N], final[N:2 * N],
                          final[2 * N:3 * N], final[3 * N:]], axis=1)
    return fe[:N_USERS], fe[N_USERS:]


# speedup vs baseline: 7.5877x; 1.0203x over previous
import functools

import jax
import jax.numpy as jnp
from jax import lax
from jax.experimental import pallas as pl
from jax.experimental.pallas import tpu as pltpu
from jax.experimental.pallas import tpu_sc as plsc

N_USERS = 25000
N_ITEMS = 25000
N = N_USERS + N_ITEMS
D = 64
DQ = 16
E = 800000
NLAYERS = 3
NSUB = 16

SLEN = 128
NSTREAM = 8
EPAD = 819200
EROWS = EPAD // SLEN
NBPW = EROWS // NSTREAM // NSUB

RCH = 125
NCH = N // RCH
NCHS = NCH // NSUB
LDW = N // NSUB

_mesh = plsc.VectorSubcoreMesh(core_axis_name="c", subcore_axis_name="s")

_f32 = jnp.float32


@functools.partial(
    pl.kernel,
    mesh=_mesh,
    compiler_params=pltpu.CompilerParams(use_tc_tiling_on_sc=False),
    out_type=[
        jax.ShapeDtypeStruct((4 * N, DQ), _f32),
        jax.ShapeDtypeStruct((4 * N, DQ), _f32),
        jax.ShapeDtypeStruct((4 * N, DQ), _f32),
        jax.ShapeDtypeStruct((4 * N, DQ), _f32),
        jax.ShapeDtypeStruct((2 * N, DQ), _f32),
    ],
    scratch_types=[
        pltpu.VMEM_SHARED((N, DQ), _f32),
        pltpu.VMEM_SHARED((N + 8, DQ), _f32),
        pltpu.VMEM((NSTREAM, SLEN), jnp.int32),
        pltpu.VMEM((NSTREAM, SLEN), jnp.int32),
        pltpu.VMEM((SLEN, DQ), _f32),
        pltpu.VMEM((SLEN, DQ), _f32),
        pltpu.VMEM((SLEN, DQ), _f32),
        pltpu.VMEM((RCH, DQ), _f32),
        pltpu.VMEM((RCH, DQ), _f32),
        pltpu.VMEM((RCH, DQ), _f32),
        pltpu.VMEM((RCH, DQ), _f32),
        pltpu.SemaphoreType.DMA,
        pltpu.SemaphoreType.DMA,
        pltpu.SemaphoreType.DMA,
        pltpu.SemaphoreType.DMA,
        pltpu.SemaphoreType.DMA,
        pltpu.SemaphoreType.DMA,
        pltpu.SemaphoreType.DMA,
        pltpu.SemaphoreType.DMA,
    ],
)
def _gcn_sc(src2, dst2, x0q, final, t1, t2, accio, rec,
            tbl_sh, acc_sh, sidx, didx, rows, rows_1, rows_2,
            xbuf, abuf, rbuf, zbuf, sem_a, sem_b,
            gs0, gs1, gs2, ss0, ss1, ss2):
    cidx = lax.axis_index("c")
    sid = lax.axis_index("s")
    cbase = cidx * N
    qbase0 = 2 * cidx * N

    zvec = jnp.zeros((16,), _f32)
    onev = jnp.ones((16,), _f32)
    gsems = (gs0, gs1, gs2)
    ssems = (ss0, ss1, ss2)
    bufs = (rows, rows_1, rows_2)

    def fill(ref, n, vec):
        @pl.loop(0, n)
        def _(r):
            ref[r, pl.ds(0, 16)] = vec

    def chunk_loop(body):
        @pl.loop(0, NCHS)
        def _(j):
            body(j, (j * NSUB + sid) * RCH)

    def zero_phase():
        fill(abuf, RCH, zvec)
        chunk_loop(lambda j, r0:
                   pltpu.sync_copy(abuf, acc_sh.at[pl.ds(r0, RCH)]))

    def load_phase(src, qb):
        r0 = sid * LDW
        pltpu.sync_copy(src.at[pl.ds(qb + r0, LDW)],
                        tbl_sh.at[pl.ds(r0, LDW)])

    def count_phase():
        fill(rows, SLEN, onev)

        @pl.loop(0, NBPW)
        def _(b):
            r0 = (sid * NBPW + b) * NSTREAM
            pltpu.sync_copy(dst2.at[pl.ds(r0, NSTREAM)], didx)
            ss = [pltpu.async_copy(rows, acc_sh.at[didx.at[k]],
                                   ssems[k % 3], add=True)
                  for k in range(NSTREAM)]
            for s in ss:
                s.wait()

    def extract_phase():
        def body(j, r0):
            pltpu.sync_copy(acc_sh.at[pl.ds(r0, RCH)], xbuf)

            @pl.loop(0, RCH)
            def _(r):
                cv = xbuf[r, pl.ds(0, 16)]
                rbuf[r, pl.ds(0, 16)] = 1.0 / jnp.maximum(cv, 1.0)

            pltpu.sync_copy(rbuf, rec.at[pl.ds(cbase + r0, RCH)])

        chunk_loop(body)

    def stream_phase():
        @pl.loop(0, NBPW)
        def _(b):
            r0 = (sid * NBPW + b) * NSTREAM
            ci = pltpu.async_copy(src2.at[pl.ds(r0, NSTREAM)], sidx, sem_a)
            cd = pltpu.async_copy(dst2.at[pl.ds(r0, NSTREAM)], didx, sem_b)
            ci.wait()
            cd.wait()
            g = [None] * NSTREAM
            s = [None] * NSTREAM
            for j in range(2):
                g[j] = pltpu.async_copy(tbl_sh.at[sidx.at[j]], bufs[j],
                                        gsems[j])
            for k in range(NSTREAM):
                g[k].wait()
                s[k] = pltpu.async_copy(bufs[k % 3], acc_sh.at[didx.at[k]],
                                        ssems[k % 3], add=True)
                j = k + 2
                if j < NSTREAM:
                    if j >= 3:
                        s[j - 3].wait()
                    g[j] = pltpu.async_copy(tbl_sh.at[sidx.at[j]],
                                            bufs[j % 3], gsems[j % 3])
            for k in range(NSTREAM - 3, NSTREAM):
                s[k].wait()

    def norm_phase(prev, nxt, qb, last, load_src=None, load_qb=0):
        if load_src is not None:
            lr0 = sid * LDW
            lcp = pltpu.async_copy(load_src.at[pl.ds(load_qb + lr0, LDW)],
                                   tbl_sh.at[pl.ds(lr0, LDW)], gs0)
        fill(zbuf, RCH, zvec)

        def body(j, r0):
            cp1 = pltpu.async_copy(prev.at[pl.ds(qb + r0, RCH)], abuf, sem_a)
            cp2 = pltpu.async_copy(rec.at[pl.ds(cbase + r0, RCH)], rbuf,
                                   sem_b)
            pltpu.sync_copy(acc_sh.at[pl.ds(r0, RCH)], xbuf)
            if load_src is not None:
                pltpu.sync_copy(zbuf, acc_sh.at[pl.ds(r0, RCH)])
            cp1.wait()
            cp2.wait()

            @pl.loop(0, RCH)
            def _(r):
                sl = pl.ds(0, 16)
                xv = xbuf[r, sl] * rbuf[r, sl]
                av = abuf[r, sl] + xv
                if last:
                    abuf[r, sl] = av * 0.25
                else:
                    xbuf[r, sl] = xv
                    abuf[r, sl] = av

            if last:
                pltpu.sync_copy(abuf, final.at[pl.ds(qb + r0, RCH)])
            else:
                pltpu.sync_copy(xbuf, nxt.at[pl.ds(qb + r0, RCH)])
                pltpu.sync_copy(abuf, accio.at[pl.ds(qb + r0, RCH)])

        chunk_loop(body)
        if load_src is not None:
            lcp.wait()

    zero_phase()
    plsc.subcore_barrier()
    count_phase()
    plsc.subcore_barrier()
    extract_phase()
    zero_phase()
    load_phase(x0q, qbase0)
    plsc.subcore_barrier()

    tbls = [x0q, t1, t2]
    prevs = [x0q, accio, accio]
    rounds = [(l, q) for l in range(NLAYERS) for q in range(2)]
    for i, (l, q) in enumerate(rounds):
        stream_phase()
        plsc.subcore_barrier()
        if i + 1 < len(rounds):
            nl, nq = rounds[i + 1]
            load_src, load_qb = tbls[nl], qbase0 + nq * N
        else:
            load_src, load_qb = None, 0
        norm_phase(prevs[l], tbls[l + 1] if l < NLAYERS - 1 else None,
                   qbase0 + q * N, last=(l == NLAYERS - 1),
                   load_src=load_src, load_qb=load_qb)
        plsc.subcore_barrier()


def kernel(user_weight, item_weight, edge_index):
    x = jnp.concatenate([user_weight, item_weight], axis=0)
    x0q = jnp.concatenate([x[:, 0:16], x[:, 16:32], x[:, 32:48], x[:, 48:64]],
                          axis=0)
    src = edge_index[0].astype(jnp.int32)
    dst = edge_index[1].astype(jnp.int32)
    pad = EPAD - E
    src2 = jnp.concatenate([src, jnp.zeros((pad,), jnp.int32)])
    dst2 = jnp.concatenate([dst, jnp.full((pad,), N, jnp.int32)])
    final, _t1, _t2, _acc, _rec = _gcn_sc(
        src2.reshape(EROWS, SLEN), dst2.reshape(EROWS, SLEN), x0q)
    fe = jnp.concatenate([final[0:N], final[N:2 * N],
                          final[2 * N:3 * N], final[3 * N:]], axis=1)
    return fe[:N_USERS], fe[N_USERS:]


# speedup vs baseline: 7.8643x; 1.0365x over previous
import functools

import jax
import jax.numpy as jnp
from jax import lax
from jax.experimental import pallas as pl
from jax.experimental.pallas import tpu as pltpu
from jax.experimental.pallas import tpu_sc as plsc

N_USERS = 25000
N_ITEMS = 25000
N = N_USERS + N_ITEMS
D = 64
DQ = 16
E = 800000
NLAYERS = 3
NSUB = 16

SLEN = 128
NSTREAM = 8
EPAD = 819200
EROWS = EPAD // SLEN
NBPW = EROWS // NSTREAM // NSUB

RCH = 250
NCH = N // RCH
NCHS = -(-NCH // NSUB)
LDW = N // NSUB

_mesh = plsc.VectorSubcoreMesh(core_axis_name="c", subcore_axis_name="s")

_f32 = jnp.float32


@functools.partial(
    pl.kernel,
    mesh=_mesh,
    compiler_params=pltpu.CompilerParams(use_tc_tiling_on_sc=False),
    out_type=[
        jax.ShapeDtypeStruct((4 * N, DQ), _f32),
        jax.ShapeDtypeStruct((4 * N, DQ), _f32),
        jax.ShapeDtypeStruct((4 * N, DQ), _f32),
        jax.ShapeDtypeStruct((4 * N, DQ), _f32),
        jax.ShapeDtypeStruct((2 * N, DQ), _f32),
    ],
    scratch_types=[
        pltpu.VMEM_SHARED((N, DQ), _f32),
        pltpu.VMEM_SHARED((N + 8, DQ), _f32),
        pltpu.VMEM((NSTREAM, SLEN), jnp.int32),
        pltpu.VMEM((NSTREAM, SLEN), jnp.int32),
        pltpu.VMEM((SLEN, DQ), _f32),
        pltpu.VMEM((SLEN, DQ), _f32),
        pltpu.VMEM((SLEN, DQ), _f32),
        pltpu.VMEM((RCH, DQ), _f32),
        pltpu.VMEM((RCH, DQ), _f32),
        pltpu.VMEM((RCH, DQ), _f32),
        pltpu.VMEM((RCH, DQ), _f32),
        pltpu.SemaphoreType.DMA,
        pltpu.SemaphoreType.DMA,
        pltpu.SemaphoreType.DMA,
        pltpu.SemaphoreType.DMA,
        pltpu.SemaphoreType.DMA,
        pltpu.SemaphoreType.DMA,
        pltpu.SemaphoreType.DMA,
        pltpu.SemaphoreType.DMA,
    ],
)
def _gcn_sc(src2, dst2, x0q, final, t1, t2, accio, rec,
            tbl_sh, acc_sh, sidx, didx, rows, rows_1, rows_2,
            xbuf, abuf, rbuf, zbuf, sem_a, sem_b,
            gs0, gs1, gs2, ss0, ss1, ss2):
    cidx = lax.axis_index("c")
    sid = lax.axis_index("s")
    cbase = cidx * N
    qbase0 = 2 * cidx * N

    zvec = jnp.zeros((16,), _f32)
    onev = jnp.ones((16,), _f32)
    gsems = (gs0, gs1, gs2)
    ssems = (ss0, ss1, ss2)
    bufs = (rows, rows_1, rows_2)

    def fill(ref, n, vec):
        @pl.loop(0, n)
        def _(r):
            ref[r, pl.ds(0, 16)] = vec

    def chunk_loop(body):
        @pl.loop(0, NCHS)
        def _(j):
            g = j * NSUB + sid

            @pl.when(g < NCH)
            def _():
                body(j, g * RCH)

    def zero_phase():
        fill(abuf, RCH, zvec)
        chunk_loop(lambda j, r0:
                   pltpu.sync_copy(abuf, acc_sh.at[pl.ds(r0, RCH)]))

    def load_phase(src, qb):
        r0 = sid * LDW
        pltpu.sync_copy(src.at[pl.ds(qb + r0, LDW)],
                        tbl_sh.at[pl.ds(r0, LDW)])

    def count_phase():
        fill(rows, SLEN, onev)

        @pl.loop(0, NBPW)
        def _(b):
            r0 = (sid * NBPW + b) * NSTREAM
            pltpu.sync_copy(dst2.at[pl.ds(r0, NSTREAM)], didx)
            ss = [pltpu.async_copy(rows, acc_sh.at[didx.at[k]],
                                   ssems[k % 3], add=True)
                  for k in range(NSTREAM)]
            for s in ss:
                s.wait()

    def extract_phase():
        def body(j, r0):
            pltpu.sync_copy(acc_sh.at[pl.ds(r0, RCH)], xbuf)

            @pl.loop(0, RCH)
            def _(r):
                cv = xbuf[r, pl.ds(0, 16)]
                rbuf[r, pl.ds(0, 16)] = 1.0 / jnp.maximum(cv, 1.0)

            pltpu.sync_copy(rbuf, rec.at[pl.ds(cbase + r0, RCH)])

        chunk_loop(body)

    def stream_phase():
        @pl.loop(0, NBPW)
        def _(b):
            r0 = (sid * NBPW + b) * NSTREAM
            ci = pltpu.async_copy(src2.at[pl.ds(r0, NSTREAM)], sidx, sem_a)
            cd = pltpu.async_copy(dst2.at[pl.ds(r0, NSTREAM)], didx, sem_b)
            ci.wait()
            cd.wait()
            g = [None] * NSTREAM
            s = [None] * NSTREAM
            for j in range(2):
                g[j] = pltpu.async_copy(tbl_sh.at[sidx.at[j]], bufs[j],
                                        gsems[j])
            for k in range(NSTREAM):
                g[k].wait()
                s[k] = pltpu.async_copy(bufs[k % 3], acc_sh.at[didx.at[k]],
                                        ssems[k % 3], add=True)
                j = k + 2
                if j < NSTREAM:
                    if j >= 3:
                        s[j - 3].wait()
                    g[j] = pltpu.async_copy(tbl_sh.at[sidx.at[j]],
                                            bufs[j % 3], gsems[j % 3])
            for k in range(NSTREAM - 3, NSTREAM):
                s[k].wait()

    def norm_phase(prev, nxt, qb, last, load_src=None, load_qb=0):
        if load_src is not None:
            lr0 = sid * LDW
            lcp = pltpu.async_copy(load_src.at[pl.ds(load_qb + lr0, LDW)],
                                   tbl_sh.at[pl.ds(lr0, LDW)], gs0)
        fill(zbuf, RCH, zvec)

        def body(j, r0):
            cp1 = pltpu.async_copy(prev.at[pl.ds(qb + r0, RCH)], abuf, sem_a)
            cp2 = pltpu.async_copy(rec.at[pl.ds(cbase + r0, RCH)], rbuf,
                                   sem_b)
            pltpu.sync_copy(acc_sh.at[pl.ds(r0, RCH)], xbuf)
            if load_src is not None:
                pltpu.sync_copy(zbuf, acc_sh.at[pl.ds(r0, RCH)])
            cp1.wait()
            cp2.wait()

            @pl.loop(0, RCH)
            def _(r):
                sl = pl.ds(0, 16)
                xv = xbuf[r, sl] * rbuf[r, sl]
                av = abuf[r, sl] + xv
                if last:
                    abuf[r, sl] = av * 0.25
                else:
                    xbuf[r, sl] = xv
                    abuf[r, sl] = av

            if last:
                pltpu.sync_copy(abuf, final.at[pl.ds(qb + r0, RCH)])
            else:
                w1 = pltpu.async_copy(xbuf, nxt.at[pl.ds(qb + r0, RCH)], ss1)
                w2 = pltpu.async_copy(abuf, accio.at[pl.ds(qb + r0, RCH)],
                                      ss2)
                w1.wait()
                w2.wait()

        chunk_loop(body)
        if load_src is not None:
            lcp.wait()

    zero_phase()
    plsc.subcore_barrier()
    count_phase()
    plsc.subcore_barrier()
    extract_phase()
    zero_phase()
    load_phase(x0q, qbase0)
    plsc.subcore_barrier()

    tbls = [x0q, t1, t2]
    prevs = [x0q, accio, accio]
    rounds = [(l, q) for l in range(NLAYERS) for q in range(2)]
    for i, (l, q) in enumerate(rounds):
        stream_phase()
        plsc.subcore_barrier()
        if i + 1 < len(rounds):
            nl, nq = rounds[i + 1]
            load_src, load_qb = tbls[nl], qbase0 + nq * N
        else:
            load_src, load_qb = None, 0
        norm_phase(prevs[l], tbls[l + 1] if l < NLAYERS - 1 else None,
                   qbase0 + q * N, last=(l == NLAYERS - 1),
                   load_src=load_src, load_qb=load_qb)
        plsc.subcore_barrier()


def kernel(user_weight, item_weight, edge_index):
    x = jnp.concatenate([user_weight, item_weight], axis=0)
    x0q = jnp.concatenate([x[:, 0:16], x[:, 16:32], x[:, 32:48], x[:, 48:64]],
                          axis=0)
    src = edge_index[0].astype(jnp.int32)
    dst = edge_index[1].astype(jnp.int32)
    pad = EPAD - E
    src2 = jnp.concatenate([src, jnp.zeros((pad,), jnp.int32)])
    dst2 = jnp.concatenate([dst, jnp.full((pad,), N, jnp.int32)])
    final, _t1, _t2, _acc, _rec = _gcn_sc(
        src2.reshape(EROWS, SLEN), dst2.reshape(EROWS, SLEN), x0q)
    fe = jnp.concatenate([final[0:N], final[N:2 * N],
                          final[2 * N:3 * N], final[3 * N:]], axis=1)
    return fe[:N_USERS], fe[N_USERS:]


# speedup vs baseline: 7.9562x; 1.0117x over previous
import functools

import jax
import jax.numpy as jnp
from jax import lax
from jax.experimental import pallas as pl
from jax.experimental.pallas import tpu as pltpu
from jax.experimental.pallas import tpu_sc as plsc

N_USERS = 25000
N_ITEMS = 25000
N = N_USERS + N_ITEMS
D = 64
DQ = 16
E = 800000
NLAYERS = 3
NSUB = 16

SLEN = 256
NSTREAM = 4
EPAD = 819200
EROWS = EPAD // SLEN
NBPW = EROWS // NSTREAM // NSUB

RCH = 250
NCH = N // RCH
NCHS = -(-NCH // NSUB)
LDW = N // NSUB

_mesh = plsc.VectorSubcoreMesh(core_axis_name="c", subcore_axis_name="s")

_f32 = jnp.float32


@functools.partial(
    pl.kernel,
    mesh=_mesh,
    compiler_params=pltpu.CompilerParams(use_tc_tiling_on_sc=False),
    out_type=[
        jax.ShapeDtypeStruct((4 * N, DQ), _f32),
        jax.ShapeDtypeStruct((4 * N, DQ), _f32),
        jax.ShapeDtypeStruct((4 * N, DQ), _f32),
        jax.ShapeDtypeStruct((4 * N, DQ), _f32),
        jax.ShapeDtypeStruct((2 * N, DQ), _f32),
    ],
    scratch_types=[
        pltpu.VMEM_SHARED((N, DQ), _f32),
        pltpu.VMEM_SHARED((N + 8, DQ), _f32),
        pltpu.VMEM((NSTREAM, SLEN), jnp.int32),
        pltpu.VMEM((NSTREAM, SLEN), jnp.int32),
        pltpu.VMEM((SLEN, DQ), _f32),
        pltpu.VMEM((SLEN, DQ), _f32),
        pltpu.VMEM((SLEN, DQ), _f32),
        pltpu.VMEM((RCH, DQ), _f32),
        pltpu.VMEM((RCH, DQ), _f32),
        pltpu.VMEM((RCH, DQ), _f32),
        pltpu.VMEM((RCH, DQ), _f32),
        pltpu.SemaphoreType.DMA,
        pltpu.SemaphoreType.DMA,
        pltpu.SemaphoreType.DMA,
        pltpu.SemaphoreType.DMA,
        pltpu.SemaphoreType.DMA,
        pltpu.SemaphoreType.DMA,
        pltpu.SemaphoreType.DMA,
        pltpu.SemaphoreType.DMA,
    ],
)
def _gcn_sc(src2, dst2, x0q, final, t1, t2, accio, rec,
            tbl_sh, acc_sh, sidx, didx, rows, rows_1, rows_2,
            xbuf, abuf, rbuf, zbuf, sem_a, sem_b,
            gs0, gs1, gs2, ss0, ss1, ss2):
    cidx = lax.axis_index("c")
    sid = lax.axis_index("s")
    cbase = cidx * N
    qbase0 = 2 * cidx * N

    zvec = jnp.zeros((16,), _f32)
    onev = jnp.ones((16,), _f32)
    gsems = (gs0, gs1, gs2)
    ssems = (ss0, ss1, ss2)
    bufs = (rows, rows_1, rows_2)

    def fill(ref, n, vec):
        @pl.loop(0, n)
        def _(r):
            ref[r, pl.ds(0, 16)] = vec

    def chunk_loop(body):
        @pl.loop(0, NCHS)
        def _(j):
            g = j * NSUB + sid

            @pl.when(g < NCH)
            def _():
                body(j, g * RCH)

    def zero_phase():
        fill(abuf, RCH, zvec)
        chunk_loop(lambda j, r0:
                   pltpu.sync_copy(abuf, acc_sh.at[pl.ds(r0, RCH)]))

    def load_phase(src, qb):
        r0 = sid * LDW
        pltpu.sync_copy(src.at[pl.ds(qb + r0, LDW)],
                        tbl_sh.at[pl.ds(r0, LDW)])

    def count_phase():
        fill(rows, SLEN, onev)

        @pl.loop(0, NBPW)
        def _(b):
            r0 = (sid * NBPW + b) * NSTREAM
            pltpu.sync_copy(dst2.at[pl.ds(r0, NSTREAM)], didx)
            ss = [pltpu.async_copy(rows, acc_sh.at[didx.at[k]],
                                   ssems[k % 3], add=True)
                  for k in range(NSTREAM)]
            for s in ss:
                s.wait()

    def extract_phase():
        def body(j, r0):
            pltpu.sync_copy(acc_sh.at[pl.ds(r0, RCH)], xbuf)

            @pl.loop(0, RCH, step=5)
            def _(r):
                for u in range(5):
                    cv = xbuf[r + u, pl.ds(0, 16)]
                    rbuf[r + u, pl.ds(0, 16)] = 1.0 / jnp.maximum(cv, 1.0)

            pltpu.sync_copy(rbuf, rec.at[pl.ds(cbase + r0, RCH)])

        chunk_loop(body)

    def stream_phase():
        @pl.loop(0, NBPW)
        def _(b):
            r0 = (sid * NBPW + b) * NSTREAM
            ci = pltpu.async_copy(src2.at[pl.ds(r0, NSTREAM)], sidx, sem_a)
            cd = pltpu.async_copy(dst2.at[pl.ds(r0, NSTREAM)], didx, sem_b)
            ci.wait()
            cd.wait()
            g = [None] * NSTREAM
            s = [None] * NSTREAM
            for j in range(2):
                g[j] = pltpu.async_copy(tbl_sh.at[sidx.at[j]], bufs[j],
                                        gsems[j])
            for k in range(NSTREAM):
                g[k].wait()
                s[k] = pltpu.async_copy(bufs[k % 3], acc_sh.at[didx.at[k]],
                                        ssems[k % 3], add=True)
                j = k + 2
                if j < NSTREAM:
                    if j >= 3:
                        s[j - 3].wait()
                    g[j] = pltpu.async_copy(tbl_sh.at[sidx.at[j]],
                                            bufs[j % 3], gsems[j % 3])
            for k in range(NSTREAM - 3, NSTREAM):
                s[k].wait()

    def norm_phase(prev, nxt, qb, last, load_src=None, load_qb=0):
        if load_src is not None:
            lr0 = sid * LDW
            lcp = pltpu.async_copy(load_src.at[pl.ds(load_qb + lr0, LDW)],
                                   tbl_sh.at[pl.ds(lr0, LDW)], gs0)
        fill(zbuf, RCH, zvec)

        def body(j, r0):
            cp1 = pltpu.async_copy(prev.at[pl.ds(qb + r0, RCH)], abuf, sem_a)
            cp2 = pltpu.async_copy(rec.at[pl.ds(cbase + r0, RCH)], rbuf,
                                   sem_b)
            pltpu.sync_copy(acc_sh.at[pl.ds(r0, RCH)], xbuf)
            if load_src is not None:
                pltpu.sync_copy(zbuf, acc_sh.at[pl.ds(r0, RCH)])
            cp1.wait()
            cp2.wait()

            @pl.loop(0, RCH, step=5)
            def _(r):
                sl = pl.ds(0, 16)
                for u in range(5):
                    xv = xbuf[r + u, sl] * rbuf[r + u, sl]
                    av = abuf[r + u, sl] + xv
                    if last:
                        abuf[r + u, sl] = av * 0.25
                    else:
                        xbuf[r + u, sl] = xv
                        abuf[r + u, sl] = av

            if last:
                pltpu.sync_copy(abuf, final.at[pl.ds(qb + r0, RCH)])
            else:
                w1 = pltpu.async_copy(xbuf, nxt.at[pl.ds(qb + r0, RCH)], ss1)
                w2 = pltpu.async_copy(abuf, accio.at[pl.ds(qb + r0, RCH)],
                                      ss2)
                w1.wait()
                w2.wait()

        chunk_loop(body)
        if load_src is not None:
            lcp.wait()

    zero_phase()
    plsc.subcore_barrier()
    count_phase()
    plsc.subcore_barrier()
    extract_phase()
    zero_phase()
    load_phase(x0q, qbase0)
    plsc.subcore_barrier()

    tbls = [x0q, t1, t2]
    prevs = [x0q, accio, accio]
    rounds = [(l, q) for l in range(NLAYERS) for q in range(2)]
    for i, (l, q) in enumerate(rounds):
        stream_phase()
        plsc.subcore_barrier()
        if i + 1 < len(rounds):
            nl, nq = rounds[i + 1]
            load_src, load_qb = tbls[nl], qbase0 + nq * N
        else:
            load_src, load_qb = None, 0
        norm_phase(prevs[l], tbls[l + 1] if l < NLAYERS - 1 else None,
                   qbase0 + q * N, last=(l == NLAYERS - 1),
                   load_src=load_src, load_qb=load_qb)
        plsc.subcore_barrier()


def kernel(user_weight, item_weight, edge_index):
    x = jnp.concatenate([user_weight, item_weight], axis=0)
    x0q = jnp.concatenate([x[:, 0:16], x[:, 16:32], x[:, 32:48], x[:, 48:64]],
                          axis=0)
    src = edge_index[0].astype(jnp.int32)
    dst = edge_index[1].astype(jnp.int32)
    pad = EPAD - E
    src2 = jnp.concatenate([src, jnp.zeros((pad,), jnp.int32)])
    dst2 = jnp.concatenate([dst, jnp.full((pad,), N, jnp.int32)])
    final, _t1, _t2, _acc, _rec = _gcn_sc(
        src2.reshape(EROWS, SLEN), dst2.reshape(EROWS, SLEN), x0q)
    fe = jnp.concatenate([final[0:N], final[N:2 * N],
                          final[2 * N:3 * N], final[3 * N:]], axis=1)
    return fe[:N_USERS], fe[N_USERS:]


# speedup vs baseline: 9.5522x; 1.2006x over previous
import functools

import jax
import jax.numpy as jnp
from jax import lax
from jax.experimental import pallas as pl
from jax.experimental.pallas import tpu as pltpu
from jax.experimental.pallas import tpu_sc as plsc

N_USERS = 25000
N_ITEMS = 25000
N = N_USERS + N_ITEMS
D = 64
DQ = 16
E = 800000
NLAYERS = 3
NSUB = 16

SLEN = 256
NSTREAM = 4
EPAD = 819200
EROWS = EPAD // SLEN
NBPW = EROWS // NSTREAM // NSUB

RCH = 250
NCH = N // RCH
NCHS = -(-NCH // NSUB)
LDW = N // NSUB

_mesh = plsc.VectorSubcoreMesh(core_axis_name="c", subcore_axis_name="s")

_f32 = jnp.float32


@functools.partial(
    pl.kernel,
    mesh=_mesh,
    compiler_params=pltpu.CompilerParams(use_tc_tiling_on_sc=False),
    out_type=[
        jax.ShapeDtypeStruct((N_USERS, D), _f32),
        jax.ShapeDtypeStruct((N_ITEMS, D), _f32),
        jax.ShapeDtypeStruct((4 * N, DQ), _f32),
        jax.ShapeDtypeStruct((4 * N, DQ), _f32),
        jax.ShapeDtypeStruct((4 * N, DQ), _f32),
        jax.ShapeDtypeStruct((2 * N, DQ), _f32),
    ],
    scratch_types=[
        pltpu.VMEM_SHARED((N, DQ), _f32),
        pltpu.VMEM_SHARED((N + 8, DQ), _f32),
        pltpu.VMEM((NSTREAM, SLEN), jnp.int32),
        pltpu.VMEM((NSTREAM, SLEN), jnp.int32),
        pltpu.VMEM((SLEN, DQ), _f32),
        pltpu.VMEM((SLEN, DQ), _f32),
        pltpu.VMEM((SLEN, DQ), _f32),
        pltpu.VMEM((RCH, DQ), _f32),
        pltpu.VMEM((RCH, DQ), _f32),
        pltpu.VMEM((RCH, DQ), _f32),
        pltpu.VMEM((RCH, DQ), _f32),
        pltpu.SemaphoreType.DMA,
        pltpu.SemaphoreType.DMA,
        pltpu.SemaphoreType.DMA,
        pltpu.SemaphoreType.DMA,
        pltpu.SemaphoreType.DMA,
        pltpu.SemaphoreType.DMA,
        pltpu.SemaphoreType.DMA,
        pltpu.SemaphoreType.DMA,
    ],
)
def _gcn_sc(src2, dst2, x0, uout, iout, t1, t2, accio, rec,
            tbl_sh, acc_sh, sidx, didx, rows, rows_1, rows_2,
            xbuf, abuf, rbuf, zbuf, sem_a, sem_b,
            gs0, gs1, gs2, ss0, ss1, ss2):
    cidx = lax.axis_index("c")
    sid = lax.axis_index("s")
    cbase = cidx * N
    qbase0 = 2 * cidx * N

    zvec = jnp.zeros((16,), _f32)
    onev = jnp.ones((16,), _f32)
    gsems = (gs0, gs1, gs2)
    ssems = (ss0, ss1, ss2)
    bufs = (rows, rows_1, rows_2)

    def fill(ref, n, vec):
        @pl.loop(0, n)
        def _(r):
            ref[r, pl.ds(0, 16)] = vec

    def chunk_loop(body):
        @pl.loop(0, NCHS)
        def _(j):
            g = j * NSUB + sid

            @pl.when(g < NCH)
            def _():
                body(j, g * RCH)

    def zero_phase():
        fill(abuf, RCH, zvec)
        chunk_loop(lambda j, r0:
                   pltpu.sync_copy(abuf, acc_sh.at[pl.ds(r0, RCH)]))

    def load_phase(slc):
        r0 = sid * LDW
        pltpu.sync_copy(slc(r0, LDW), tbl_sh.at[pl.ds(r0, LDW)])

    def count_phase():
        fill(rows, SLEN, onev)

        @pl.loop(0, NBPW)
        def _(b):
            r0 = (sid * NBPW + b) * NSTREAM
            pltpu.sync_copy(dst2.at[pl.ds(r0, NSTREAM)], didx)
            ss = [pltpu.async_copy(rows, acc_sh.at[didx.at[k]],
                                   ssems[k % 3], add=True)
                  for k in range(NSTREAM)]
            for s in ss:
                s.wait()

    def extract_phase():
        def body(j, r0):
            pltpu.sync_copy(acc_sh.at[pl.ds(r0, RCH)], xbuf)

            @pl.loop(0, RCH, step=5)
            def _(r):
                for u in range(5):
                    cv = xbuf[r + u, pl.ds(0, 16)]
                    rbuf[r + u, pl.ds(0, 16)] = 1.0 / jnp.maximum(cv, 1.0)

            pltpu.sync_copy(rbuf, rec.at[pl.ds(cbase + r0, RCH)])

        chunk_loop(body)

    def stream_phase():
        @pl.loop(0, NBPW)
        def _(b):
            r0 = (sid * NBPW + b) * NSTREAM
            ci = pltpu.async_copy(src2.at[pl.ds(r0, NSTREAM)], sidx, sem_a)
            cd = pltpu.async_copy(dst2.at[pl.ds(r0, NSTREAM)], didx, sem_b)
            ci.wait()
            cd.wait()
            g = [None] * NSTREAM
            s = [None] * NSTREAM
            for j in range(2):
                g[j] = pltpu.async_copy(tbl_sh.at[sidx.at[j]], bufs[j],
                                        gsems[j])
            for k in range(NSTREAM):
                g[k].wait()
                s[k] = pltpu.async_copy(bufs[k % 3], acc_sh.at[didx.at[k]],
                                        ssems[k % 3], add=True)
                j = k + 2
                if j < NSTREAM:
                    if j >= 3:
                        s[j - 3].wait()
                    g[j] = pltpu.async_copy(tbl_sh.at[sidx.at[j]],
                                            bufs[j % 3], gsems[j % 3])
            for k in range(NSTREAM - 3, NSTREAM):
                s[k].wait()

    def norm_phase(prev_slc, nxt, qb, qcol, last, load_slc=None):
        if load_slc is not None:
            lr0 = sid * LDW
            lcp = pltpu.async_copy(load_slc(lr0, LDW),
                                   tbl_sh.at[pl.ds(lr0, LDW)], gs0)
        fill(zbuf, RCH, zvec)

        def body(j, r0):
            cp1 = pltpu.async_copy(prev_slc(r0, RCH), abuf, sem_a)
            cp2 = pltpu.async_copy(rec.at[pl.ds(cbase + r0, RCH)], rbuf,
                                   sem_b)
            pltpu.sync_copy(acc_sh.at[pl.ds(r0, RCH)], xbuf)
            if load_slc is not None:
                pltpu.sync_copy(zbuf, acc_sh.at[pl.ds(r0, RCH)])
            cp1.wait()
            cp2.wait()

            @pl.loop(0, RCH, step=5)
            def _(r):
                sl = pl.ds(0, 16)
                for u in range(5):
                    xv = xbuf[r + u, sl] * rbuf[r + u, sl]
                    av = abuf[r + u, sl] + xv
                    if last:
                        abuf[r + u, sl] = av * 0.25
                    else:
                        xbuf[r + u, sl] = xv
                        abuf[r + u, sl] = av

            if last:
                @pl.when(r0 < N_USERS)
                def _():
                    pltpu.sync_copy(
                        abuf, uout.at[pl.ds(r0, RCH), pl.ds(qcol, DQ)])

                @pl.when(r0 >= N_USERS)
                def _():
                    pltpu.sync_copy(
                        abuf, iout.at[pl.ds(r0 - N_USERS, RCH),
                                      pl.ds(qcol, DQ)])
            else:
                w1 = pltpu.async_copy(xbuf, nxt.at[pl.ds(qb + r0, RCH)], ss1)
                w2 = pltpu.async_copy(abuf, accio.at[pl.ds(qb + r0, RCH)],
                                      ss2)
                w1.wait()
                w2.wait()

        chunk_loop(body)
        if load_slc is not None:
            lcp.wait()

    def x0_slc(q):
        qcol = (2 * cidx + q) * DQ
        return lambda r0, ln: x0.at[pl.ds(r0, ln), pl.ds(qcol, DQ)]

    def tbl_slc(t, q):
        qb = qbase0 + q * N
        return lambda r0, ln: t.at[pl.ds(qb + r0, ln)]

    zero_phase()
    plsc.subcore_barrier()
    count_phase()
    plsc.subcore_barrier()
    extract_phase()
    zero_phase()
    load_phase(x0_slc(0))
    plsc.subcore_barrier()

    rounds = [(l, q) for l in range(NLAYERS) for q in range(2)]

    def in_slc(l, q):
        return x0_slc(q) if l == 0 else tbl_slc([None, t1, t2][l], q)

    for i, (l, q) in enumerate(rounds):
        stream_phase()
        plsc.subcore_barrier()
        if i + 1 < len(rounds):
            nl, nq = rounds[i + 1]
            load_slc = in_slc(nl, nq)
        else:
            load_slc = None
        prev_slc = x0_slc(q) if l == 0 else tbl_slc(accio, q)
        norm_phase(prev_slc, [t1, t2, None][l],
                   qbase0 + q * N, (2 * cidx + q) * DQ,
                   last=(l == NLAYERS - 1), load_slc=load_slc)
        plsc.subcore_barrier()


def kernel(user_weight, item_weight, edge_index):
    x = jnp.concatenate([user_weight, item_weight], axis=0)
    src = edge_index[0].astype(jnp.int32)
    dst = edge_index[1].astype(jnp.int32)
    pad = EPAD - E
    src2 = jnp.concatenate([src, jnp.zeros((pad,), jnp.int32)])
    dst2 = jnp.concatenate([dst, jnp.full((pad,), N, jnp.int32)])
    uout, iout, _t1, _t2, _acc, _rec = _gcn_sc(
        src2.reshape(EROWS, SLEN), dst2.reshape(EROWS, SLEN), x)
    return uout, iout
